# speedup vs baseline: 8.7899x; 8.7899x over previous
import math

import jax
import jax.numpy as jnp
from jax.experimental import pallas as pl
from jax.experimental.pallas import tpu as pltpu

_SIZE = 32768
_N = 2048
_SMOOTHING = 0.1
_CONF = 1.0 - _SMOOTHING
_EPS = _SMOOTHING / (_SIZE - 2)
_C_CONST = _CONF * math.log(_CONF) + _SMOOTHING * math.log(_EPS)
_CBLK = 2048


def _loss_body(t_ref, x_ref, out_ref):
    j = pl.program_id(0)
    t = t_ref[...]
    mrow = (t != 0).astype(jnp.float32)
    xb = x_ref[...]
    cols = j * _CBLK + jax.lax.broadcasted_iota(jnp.int32, (_N, _CBLK), 1)
    hit = cols == t
    rs = jnp.sum(xb, axis=1, keepdims=True)
    gv = jnp.sum(jnp.where(hit, xb, 0.0), axis=1, keepdims=True)
    acc = jnp.sum(mrow * (-_EPS * rs - (_CONF - _EPS) * gv))

    @pl.when(j == 0)
    def _init():
        k_valid = jnp.sum(mrow)
        x0 = jnp.sum(mrow * xb[:, 0:1])
        out_ref[0, 0] = _C_CONST * k_valid + _EPS * x0

    out_ref[0, 0] += acc


def kernel(x, target):
    t2d = target.astype(jnp.int32).reshape(_N, 1)
    out = pl.pallas_call(
        _loss_body,
        grid=(_SIZE // _CBLK,),
        in_specs=[
            pl.BlockSpec((_N, 1), lambda j: (0, 0)),
            pl.BlockSpec((_N, _CBLK), lambda j: (0, j)),
        ],
        out_specs=pl.BlockSpec(
            (1, 1), lambda j: (0, 0), memory_space=pltpu.SMEM
        ),
        out_shape=jax.ShapeDtypeStruct((1, 1), jnp.float32),
    )(t2d, x)
    return out[0, 0]
